# baseline (speedup 1.0000x reference)
import jax
import jax.numpy as jnp
from jax import lax
from jax.experimental import pallas as pl
from jax.experimental.pallas import tpu as pltpu
from jax.experimental.pallas import tpu_sc as plsc

_B = 1024
_S = 77
_D = 1024
_L = 16
_NC = 2
_NS = 16
_NW = _NC * _NS
_BB = 4
_CH = 8
_PHR = 40
_IPW = _BB * _S * _CH


def _body(ids_hbm, tok_hbm, pos_hbm, out_hbm,
          idx_v, pos_v,
          g0, g1, g2, g3, w0, w1, w2, w3,
          gs0, gs1, gs2, gs3, ws0, ws1, ws2, ws3):
    gbufs = (g0, g1, g2, g3)
    wbufs = (w0, w1, w2, w3)
    gsems = (gs0, gs1, gs2, gs3)
    wsems = (ws0, ws1, ws2, ws3)
    wid = lax.axis_index("s") * _NC + lax.axis_index("c")
    b0 = wid * (_BB * _CH)
    pltpu.sync_copy(ids_hbm.at[pl.ds(wid * _IPW, _IPW)], idx_v)

    def g_desc(s, bb):
        off = pl.multiple_of(bb * (_S * _CH) + s * _CH, 8)
        return pltpu.make_async_copy(
            tok_hbm.at[idx_v.at[pl.ds(off, _CH)]], gbufs[bb], gsems[bb])

    def w_desc(s, bb):
        return pltpu.make_async_copy(
            wbufs[bb], out_hbm.at[s, pl.ds(b0 + bb * _CH, _CH)], wsems[bb])

    def add_unit(srow, bb):
        gbuf, wbuf = gbufs[bb], wbufs[bb]

        @plsc.parallel_loop(0, _D, step=_L, unroll=4)
        def _o_body(o):
            sl = pl.ds(pl.multiple_of(o, _L), _L)
            pv = pos_v[srow, sl]
            for jj in range(_CH):
                wbuf[jj, sl] = gbuf[jj, sl] + pv

    def unit(ph, s, bb, wait_write, start_gather):
        if wait_write:
            w_desc(s, bb).wait()
        g_desc(s, bb).wait()
        add_unit(s - ph * _PHR, bb)
        w_desc(s, bb).start()
        if start_gather:
            g_desc(s + 1, bb).start()

    pltpu.sync_copy(pos_hbm.at[pl.ds(0, _PHR)], pos_v)
    for bb in range(_BB):
        g_desc(0, bb).start()
    for bb in range(_BB):
        unit(0, 0, bb, wait_write=False, start_gather=True)

    def s_body(ph):
        def body(s, carry):
            for bb in range(_BB):
                unit(ph, s, bb, wait_write=True, start_gather=True)
            return carry
        return body

    lax.fori_loop(1, _PHR, s_body(0), 0)

    pltpu.sync_copy(pos_hbm.at[pl.ds(_PHR, _PHR)], pos_v)
    lax.fori_loop(_PHR, _S - 1, s_body(1), 0)

    for bb in range(_BB):
        unit(1, _S - 1, bb, wait_write=True, start_gather=False)
    for bb in range(_BB):
        w_desc(_S - 1, bb).wait()


_embed = pl.kernel(
    _body,
    out_type=jax.ShapeDtypeStruct((_S, _B, _D), jnp.float32),
    mesh=plsc.VectorSubcoreMesh(core_axis_name="c", subcore_axis_name="s"),
    scratch_types=[
        pltpu.VMEM((_IPW,), jnp.int32),
        pltpu.VMEM((_PHR, _D), jnp.float32),
    ] + [pltpu.VMEM((_CH, _D), jnp.float32) for _ in range(8)]
      + [pltpu.SemaphoreType.DMA for _ in range(8)],
)


def kernel(input_ids, token_table, position_table):
    ids = (input_ids.astype(jnp.int32)
           .reshape(_B // _CH, _CH, _S).transpose(0, 2, 1).reshape(-1))
    pos = jnp.pad(position_table, ((0, 2 * _PHR - _S), (0, 0)))
    out = _embed(ids, token_table, pos)
    return out.transpose(1, 0, 2)
